# baseline (speedup 1.0000x reference)
import functools

import jax
import jax.numpy as jnp
from jax.experimental import pallas as pl

F32 = jnp.float32


def _deg_body(nsteps, f_ref, t_ref, i_ref, df_ref, dt_ref, di_ref, rs_ref):
    k = pl.program_id(0)
    f = f_ref[...]
    t = t_ref[...]
    i = i_ref[...]
    rs_ref[...] = jnp.sum(t, axis=1, keepdims=True)

    @pl.when(k == 0)
    def _():
        df_ref[...] = jnp.zeros_like(df_ref)
        dt_ref[...] = jnp.zeros_like(dt_ref)
        di_ref[...] = jnp.zeros_like(di_ref)

    df_ref[...] += jnp.sum(f, axis=0, keepdims=True)
    dt_ref[...] += jnp.sum(t, axis=0, keepdims=True)
    di_ref[...] += jnp.sum(i, axis=0, keepdims=True)

    @pl.when(k == nsteps - 1)
    def _():
        for r in (df_ref, dt_ref, di_ref):
            s = r[...]
            r[...] = jnp.where(s > 0, jax.lax.rsqrt(s), 0.0)


def _degrees(adj_f, adj_t, adj_i, bk=256):
    n = adj_f.shape[0]
    nk = n // bk
    return pl.pallas_call(
        functools.partial(_deg_body, nk),
        grid=(nk,),
        in_specs=[pl.BlockSpec((bk, n), lambda k: (k, 0))] * 3,
        out_specs=[pl.BlockSpec((1, n), lambda k: (0, 0))] * 3
        + [pl.BlockSpec((bk, 1), lambda k: (k, 0))],
        out_shape=[jax.ShapeDtypeStruct((1, n), F32)] * 3
        + [jax.ShapeDtypeStruct((n, 1), F32)],
    )(adj_f, adj_t, adj_i)


def _mmT_body(nk, a_ref, y_ref, dis_ref, b_ref, o_ref):
    k = pl.program_id(1)
    acc = jax.lax.dot_general(
        a_ref[...], y_ref[...], (((0,), (0,)), ((), ())),
        preferred_element_type=F32)

    @pl.when(k == 0)
    def _():
        o_ref[...] = acc

    @pl.when(k > 0)
    def _():
        o_ref[...] += acc

    @pl.when(k == nk - 1)
    def _():
        o_ref[...] = o_ref[...] * dis_ref[...] + b_ref[...]


def _gcn_mmT(A, Y, dis_col, bias_row, bi=512, bk=512):
    n = A.shape[0]
    w = Y.shape[1]
    gi, gk = n // bi, n // bk
    return pl.pallas_call(
        functools.partial(_mmT_body, gk),
        grid=(gi, gk),
        in_specs=[
            pl.BlockSpec((bk, bi), lambda i, k: (k, i)),
            pl.BlockSpec((bk, w), lambda i, k: (k, 0)),
            pl.BlockSpec((bi, 1), lambda i, k: (i, 0)),
            pl.BlockSpec((1, w), lambda i, k: (0, 0)),
        ],
        out_specs=pl.BlockSpec((bi, w), lambda i, k: (i, 0)),
        out_shape=jax.ShapeDtypeStruct((n, w), F32),
    )(A, Y, dis_col, bias_row)


def _prep1_body(feat_ref, feata_ref, w1_ref, df_ref, dt_ref, di_ref,
                yf_ref, yt_ref, ya_ref, yi_ref):
    w1 = w1_ref[...]
    p = jnp.dot(feat_ref[...], w1, preferred_element_type=F32)
    pa = jnp.dot(feata_ref[...], w1, preferred_element_type=F32)
    yf_ref[...] = df_ref[...] * p
    yt_ref[...] = dt_ref[...] * p
    ya_ref[...] = dt_ref[...] * pa
    yi_ref[...] = di_ref[...] * p


def _bn_relu(h, gamma, beta):
    m = jnp.mean(h, axis=0, keepdims=True)
    c = h - m
    v = jnp.mean(c * c, axis=0, keepdims=True)
    return jnp.maximum(c * jax.lax.rsqrt(v + 1e-5) * gamma + beta, 0.0)


def _mid_body(hf_ref, hta_ref, hi_ref, g_ref, b_ref, w2_ref,
              df_ref, dt_ref, di_ref, yf_ref, yt_ref, ya_ref, yi_ref):
    g = g_ref[...]
    b = b_ref[...]
    w2 = w2_ref[...]
    dh = w2.shape[0]

    def stage(h, dis):
        return dis * jnp.dot(_bn_relu(h, g, b), w2, preferred_element_type=F32)

    yf_ref[...] = stage(hf_ref[...], df_ref[...])
    yt_ref[...] = stage(hta_ref[:, :dh], dt_ref[...])
    ya_ref[...] = stage(hta_ref[:, dh:], dt_ref[...])
    yi_ref[...] = stage(hi_ref[...], di_ref[...])


def _readout_body(nk, a_ref, y_ref, rs_ref, o_ref):
    k = pl.program_id(1)
    acc = jnp.dot(a_ref[...], y_ref[...], preferred_element_type=F32)

    @pl.when(k == 0)
    def _():
        o_ref[...] = acc

    @pl.when(k > 0)
    def _():
        o_ref[...] += acc

    @pl.when(k == nk - 1)
    def _():
        gg = o_ref[...] / rs_ref[...]
        nrm = jnp.sqrt(jnp.sum(gg * gg, axis=1, keepdims=True))
        gg = gg / jnp.maximum(nrm, 1e-12)
        o_ref[...] = jax.nn.sigmoid(gg)


def _readout(adj_t, h1_t, rs_t, bi=512, bk=512):
    n = adj_t.shape[0]
    w = h1_t.shape[1]
    gi, gk = n // bi, n // bk
    return pl.pallas_call(
        functools.partial(_readout_body, gk),
        grid=(gi, gk),
        in_specs=[
            pl.BlockSpec((bi, bk), lambda i, k: (i, k)),
            pl.BlockSpec((bk, w), lambda i, k: (k, 0)),
            pl.BlockSpec((bi, 1), lambda i, k: (i, 0)),
        ],
        out_specs=pl.BlockSpec((bi, w), lambda i, k: (i, 0)),
        out_shape=jax.ShapeDtypeStruct((n, w), F32),
    )(adj_t, h1_t, rs_t)


def _att_body(hf_ref, ht_ref, ha_ref, hi_ref, g_ref, wb_ref, bb_ref,
              wa1_ref, ba1_ref, wa2_ref,
              sc1_ref, sc2_ref, hid_ref, bf_ref, bt_ref, bi_ref):
    hf = hf_ref[...]
    ht = ht_ref[...]
    ha = ha_ref[...]
    hi = hi_ref[...]
    g = g_ref[...]
    t = jax.lax.dot_general(
        g, wb_ref[...], (((1,), (1,)), ((), ())), preferred_element_type=F32)
    sc1_ref[...] = jnp.sum(ht * t, axis=1, keepdims=True) + bb_ref[...]
    sc2_ref[...] = jnp.sum(ha * t, axis=1, keepdims=True) + bb_ref[...]

    wa1 = wa1_ref[...]
    ba1 = ba1_ref[...]
    wa2 = wa2_ref[...]

    def score(h):
        e = jnp.tanh(jnp.dot(h, wa1, preferred_element_type=F32) + ba1)
        return jnp.sum(e * wa2, axis=1, keepdims=True)

    ef = score(hf)
    et = score(ht)
    ei = score(hi)
    mx = jnp.maximum(jnp.maximum(ef, et), ei)
    xf = jnp.exp(ef - mx)
    xt = jnp.exp(et - mx)
    xi = jnp.exp(ei - mx)
    s = xf + xt + xi
    bf = xf / s
    bt = xt / s
    bi = xi / s
    bf_ref[...] = bf
    bt_ref[...] = bt
    bi_ref[...] = bi
    hid_ref[...] = bf * hf + bt * ht + bi * hi


def _wadj_body(nsteps, f_ref, t_ref, i_ref, bf_ref, bt_ref, bi_ref,
               a_ref, cs_ref, rs_ref):
    k = pl.program_id(0)
    blk = (bf_ref[...] * f_ref[...] + bt_ref[...] * t_ref[...]
           + bi_ref[...] * i_ref[...])
    a_ref[...] = blk
    rs_ref[...] = jnp.sum(blk, axis=1, keepdims=True)

    @pl.when(k == 0)
    def _():
        cs_ref[...] = jnp.zeros_like(cs_ref)

    cs_ref[...] += jnp.sum(blk, axis=0, keepdims=True)


def _weighted_adj(adj_f, adj_t, adj_i, bf, bt, bi, bk=256):
    n = adj_f.shape[0]
    nk = n // bk
    return pl.pallas_call(
        functools.partial(_wadj_body, nk),
        grid=(nk,),
        in_specs=[pl.BlockSpec((bk, n), lambda k: (k, 0))] * 3
        + [pl.BlockSpec((bk, 1), lambda k: (k, 0))] * 3,
        out_specs=[
            pl.BlockSpec((bk, n), lambda k: (k, 0)),
            pl.BlockSpec((1, n), lambda k: (0, 0)),
            pl.BlockSpec((bk, 1), lambda k: (k, 0)),
        ],
        out_shape=[
            jax.ShapeDtypeStruct((n, n), F32),
            jax.ShapeDtypeStruct((1, n), F32),
            jax.ShapeDtypeStruct((n, 1), F32),
        ],
    )(adj_f, adj_t, adj_i, bf, bt, bi)


def _dec_in_body(cs_ref, rs_ref, hid_ref, w3_ref, y3_ref, dis_ref):
    dw = (cs_ref[...] + rs_ref[...]) / 3.0
    dis = jnp.where(dw > 0, jax.lax.rsqrt(dw), 0.0)
    dis_ref[...] = dis
    y3_ref[...] = dis * jnp.dot(hid_ref[...], w3_ref[...],
                                preferred_element_type=F32)


def _sym_body(nk, relu, a1_ref, a2_ref, y_ref, dis_ref, b_ref, o_ref):
    k = pl.program_id(1)
    y = y_ref[...]
    acc = jax.lax.dot_general(
        a1_ref[...], y, (((1,), (0,)), ((), ())), preferred_element_type=F32)
    acc += jax.lax.dot_general(
        a2_ref[...], y, (((0,), (0,)), ((), ())), preferred_element_type=F32)

    @pl.when(k == 0)
    def _():
        o_ref[...] = acc

    @pl.when(k > 0)
    def _():
        o_ref[...] += acc

    @pl.when(k == nk - 1)
    def _():
        r = dis_ref[...] * (o_ref[...] * (1.0 / 3.0)) + b_ref[...]
        if relu:
            r = jnp.maximum(r, 0.0)
        o_ref[...] = r


def _sym_mm(adj123, Y, dis_col, bias_row, relu, bi=512, bk=512):
    n = adj123.shape[0]
    w = Y.shape[1]
    gi, gk = n // bi, n // bk
    return pl.pallas_call(
        functools.partial(_sym_body, gk, relu),
        grid=(gi, gk),
        in_specs=[
            pl.BlockSpec((bi, bk), lambda i, k: (i, k)),
            pl.BlockSpec((bk, bi), lambda i, k: (k, i)),
            pl.BlockSpec((bk, w), lambda i, k: (k, 0)),
            pl.BlockSpec((bi, 1), lambda i, k: (i, 0)),
            pl.BlockSpec((1, w), lambda i, k: (0, 0)),
        ],
        out_specs=pl.BlockSpec((bi, w), lambda i, k: (i, 0)),
        out_shape=jax.ShapeDtypeStruct((n, w), F32),
    )(adj123, adj123, Y, dis_col, bias_row)


def _dec_mid_body(h_ref, g_ref, b_ref, w4_ref, dis_ref, y4_ref):
    y4_ref[...] = dis_ref[...] * jnp.dot(
        _bn_relu(h_ref[...], g_ref[...], b_ref[...]), w4_ref[...],
        preferred_element_type=F32)


def _simple_call(body, out_shapes, *args):
    return pl.pallas_call(body, out_shape=out_shapes)(*args)


def kernel(feat, feat_a, adj_f, adj_t, adj_i, W1, b1, W2, b2, W3, b3, W4, b4,
           bn_gamma, bn_beta, Wa1, ba1, Wa2, Wb, bb):
    n = adj_f.shape[0]
    din = W1.shape[0]
    dh = W1.shape[1]
    dout = W2.shape[1]

    gamma = bn_gamma.reshape(1, dh)
    beta = bn_beta.reshape(1, dh)
    b1r = b1.reshape(1, dh)
    b2r = b2.reshape(1, dout)
    b3r = b3.reshape(1, dh)
    b4r = b4.reshape(1, din)
    ba1r = ba1.reshape(1, -1)
    wa2r = Wa2.reshape(1, -1)
    bbr = bb.reshape(1, 1)
    wb0 = Wb[0]

    dis_f, dis_t, dis_i, rs_t = _degrees(adj_f, adj_t, adj_i)
    dis_f = dis_f.reshape(n, 1)
    dis_t = dis_t.reshape(n, 1)
    dis_i = dis_i.reshape(n, 1)

    sds = jax.ShapeDtypeStruct((n, dh), F32)
    y1_f, y1_t, y1_a, y1_i = _simple_call(
        _prep1_body, [sds, sds, sds, sds],
        feat, feat_a, W1, dis_f, dis_t, dis_i)
    y1_ta = jnp.concatenate([y1_t, y1_a], axis=1)

    h1a_f = _gcn_mmT(adj_f, y1_f, dis_f, b1r)
    h1a_ta = _gcn_mmT(adj_t, y1_ta, dis_t, jnp.concatenate([b1r, b1r], axis=1))
    h1a_i = _gcn_mmT(adj_i, y1_i, dis_i, b1r)

    sds2 = jax.ShapeDtypeStruct((n, dout), F32)
    y2_f, y2_t, y2_a, y2_i = _simple_call(
        _mid_body, [sds2, sds2, sds2, sds2],
        h1a_f, h1a_ta, h1a_i, gamma, beta, W2, dis_f, dis_t, dis_i)
    y2_ta = jnp.concatenate([y2_t, y2_a], axis=1)

    h1_f = _gcn_mmT(adj_f, y2_f, dis_f, b2r)
    h1_ta = _gcn_mmT(adj_t, y2_ta, dis_t, jnp.concatenate([b2r, b2r], axis=1))
    h1_i = _gcn_mmT(adj_i, y2_i, dis_i, b2r)
    h1_t = h1_ta[:, :dout]
    h1_a = h1_ta[:, dout:]

    g = _readout(adj_t, h1_t, rs_t)

    col = jax.ShapeDtypeStruct((n, 1), F32)
    sc1, sc2, hiden_emb, att_f, att_t, att_i = _simple_call(
        _att_body,
        [col, col, jax.ShapeDtypeStruct((n, dout), F32), col, col, col],
        h1_f, h1_t, h1_a, h1_i, g, wb0, bbr, Wa1, ba1r, wa2r)
    ret = jnp.concatenate([sc1, sc2], axis=1)

    adj123, cs123, rs123 = _weighted_adj(adj_f, adj_t, adj_i,
                                         att_f, att_t, att_i)

    y3, dis_w = _simple_call(
        _dec_in_body,
        [jax.ShapeDtypeStruct((n, dh), F32), jax.ShapeDtypeStruct((n, 1), F32)],
        cs123.reshape(n, 1), rs123, hiden_emb, W3)

    h2a = _sym_mm(adj123, y3, dis_w, b3r, relu=False)

    y4 = _simple_call(
        _dec_mid_body, jax.ShapeDtypeStruct((n, din), F32),
        h2a, gamma, beta, W4, dis_w)

    h2 = _sym_mm(adj123, y4, dis_w, b4r, relu=True)

    return (hiden_emb, h2, ret)


# speedup vs baseline: 1.1556x; 1.1556x over previous
import functools

import jax
import jax.numpy as jnp
from jax.experimental import pallas as pl

F32 = jnp.float32
BF16 = jnp.bfloat16


def _deg_body(nsteps, f_ref, t_ref, i_ref,
              df_ref, dt_ref, di_ref, rs_ref, fb_ref, tb_ref, ib_ref):
    k = pl.program_id(0)
    f = f_ref[...]
    t = t_ref[...]
    i = i_ref[...]
    fb_ref[...] = f.astype(BF16)
    tb_ref[...] = t.astype(BF16)
    ib_ref[...] = i.astype(BF16)
    rs_ref[...] = jnp.sum(t, axis=1, keepdims=True)

    @pl.when(k == 0)
    def _():
        df_ref[...] = jnp.zeros_like(df_ref)
        dt_ref[...] = jnp.zeros_like(dt_ref)
        di_ref[...] = jnp.zeros_like(di_ref)

    df_ref[...] += jnp.sum(f, axis=0, keepdims=True)
    dt_ref[...] += jnp.sum(t, axis=0, keepdims=True)
    di_ref[...] += jnp.sum(i, axis=0, keepdims=True)

    @pl.when(k == nsteps - 1)
    def _():
        for r in (df_ref, dt_ref, di_ref):
            s = r[...]
            r[...] = jnp.where(s > 0, jax.lax.rsqrt(s), 0.0)


def _degrees(adj_f, adj_t, adj_i, bk=256):
    n = adj_f.shape[0]
    nk = n // bk
    return pl.pallas_call(
        functools.partial(_deg_body, nk),
        grid=(nk,),
        in_specs=[pl.BlockSpec((bk, n), lambda k: (k, 0))] * 3,
        out_specs=[pl.BlockSpec((1, n), lambda k: (0, 0))] * 3
        + [pl.BlockSpec((bk, 1), lambda k: (k, 0))]
        + [pl.BlockSpec((bk, n), lambda k: (k, 0))] * 3,
        out_shape=[jax.ShapeDtypeStruct((1, n), F32)] * 3
        + [jax.ShapeDtypeStruct((n, 1), F32)]
        + [jax.ShapeDtypeStruct((n, n), BF16)] * 3,
    )(adj_f, adj_t, adj_i)


def _mmT_body(nk, a_ref, y_ref, dis_ref, b_ref, o_ref):
    k = pl.program_id(1)
    acc = jax.lax.dot_general(
        a_ref[...], y_ref[...], (((0,), (0,)), ((), ())),
        preferred_element_type=F32)

    @pl.when(k == 0)
    def _():
        o_ref[...] = acc

    @pl.when(k > 0)
    def _():
        o_ref[...] += acc

    @pl.when(k == nk - 1)
    def _():
        o_ref[...] = o_ref[...] * dis_ref[...] + b_ref[...]


def _gcn_mmT(A, Y, dis_col, bias_row, bi=512, bk=512):
    n = A.shape[0]
    w = Y.shape[1]
    gi, gk = n // bi, n // bk
    return pl.pallas_call(
        functools.partial(_mmT_body, gk),
        grid=(gi, gk),
        in_specs=[
            pl.BlockSpec((bk, bi), lambda i, k: (k, i)),
            pl.BlockSpec((bk, w), lambda i, k: (k, 0)),
            pl.BlockSpec((bi, 1), lambda i, k: (i, 0)),
            pl.BlockSpec((1, w), lambda i, k: (0, 0)),
        ],
        out_specs=pl.BlockSpec((bi, w), lambda i, k: (i, 0)),
        out_shape=jax.ShapeDtypeStruct((n, w), F32),
    )(A, Y, dis_col, bias_row)


def _prep1_body(feat_ref, feata_ref, w1_ref, df_ref, dt_ref, di_ref,
                yf_ref, yt_ref, ya_ref, yi_ref):
    w1 = w1_ref[...]
    p = jnp.dot(feat_ref[...], w1, preferred_element_type=F32)
    pa = jnp.dot(feata_ref[...], w1, preferred_element_type=F32)
    yf_ref[...] = (df_ref[...] * p).astype(BF16)
    yt_ref[...] = (dt_ref[...] * p).astype(BF16)
    ya_ref[...] = (dt_ref[...] * pa).astype(BF16)
    yi_ref[...] = (di_ref[...] * p).astype(BF16)


def _bn_relu(h, gamma, beta):
    m = jnp.mean(h, axis=0, keepdims=True)
    c = h - m
    v = jnp.mean(c * c, axis=0, keepdims=True)
    return jnp.maximum(c * jax.lax.rsqrt(v + 1e-5) * gamma + beta, 0.0)


def _mid_body(hf_ref, hta_ref, hi_ref, g_ref, b_ref, w2_ref,
              df_ref, dt_ref, di_ref, yf_ref, yt_ref, ya_ref, yi_ref):
    g = g_ref[...]
    b = b_ref[...]
    w2 = w2_ref[...]
    dh = w2.shape[0]

    def stage(h, dis):
        return (dis * jnp.dot(_bn_relu(h, g, b), w2,
                              preferred_element_type=F32)).astype(BF16)

    yf_ref[...] = stage(hf_ref[...], df_ref[...])
    yt_ref[...] = stage(hta_ref[:, :dh], dt_ref[...])
    ya_ref[...] = stage(hta_ref[:, dh:], dt_ref[...])
    yi_ref[...] = stage(hi_ref[...], di_ref[...])


def _readout_body(nk, a_ref, y_ref, rs_ref, o_ref):
    k = pl.program_id(1)
    acc = jnp.dot(a_ref[...], y_ref[...].astype(BF16),
                  preferred_element_type=F32)

    @pl.when(k == 0)
    def _():
        o_ref[...] = acc

    @pl.when(k > 0)
    def _():
        o_ref[...] += acc

    @pl.when(k == nk - 1)
    def _():
        gg = o_ref[...] / rs_ref[...]
        nrm = jnp.sqrt(jnp.sum(gg * gg, axis=1, keepdims=True))
        gg = gg / jnp.maximum(nrm, 1e-12)
        o_ref[...] = jax.nn.sigmoid(gg)


def _readout(adj_t, h1_t, rs_t, bi=512, bk=512):
    n = adj_t.shape[0]
    w = h1_t.shape[1]
    gi, gk = n // bi, n // bk
    return pl.pallas_call(
        functools.partial(_readout_body, gk),
        grid=(gi, gk),
        in_specs=[
            pl.BlockSpec((bi, bk), lambda i, k: (i, k)),
            pl.BlockSpec((bk, w), lambda i, k: (k, 0)),
            pl.BlockSpec((bi, 1), lambda i, k: (i, 0)),
        ],
        out_specs=pl.BlockSpec((bi, w), lambda i, k: (i, 0)),
        out_shape=jax.ShapeDtypeStruct((n, w), F32),
    )(adj_t, h1_t, rs_t)


def _att_body(hf_ref, ht_ref, ha_ref, hi_ref, g_ref, wb_ref, bb_ref,
              wa1_ref, ba1_ref, wa2_ref,
              sc1_ref, sc2_ref, hid_ref, bf_ref, bt_ref, bi_ref):
    hf = hf_ref[...]
    ht = ht_ref[...]
    ha = ha_ref[...]
    hi = hi_ref[...]
    g = g_ref[...]
    t = jax.lax.dot_general(
        g, wb_ref[...], (((1,), (1,)), ((), ())), preferred_element_type=F32)
    sc1_ref[...] = jnp.sum(ht * t, axis=1, keepdims=True) + bb_ref[...]
    sc2_ref[...] = jnp.sum(ha * t, axis=1, keepdims=True) + bb_ref[...]

    wa1 = wa1_ref[...]
    ba1 = ba1_ref[...]
    wa2 = wa2_ref[...]

    def score(h):
        e = jnp.tanh(jnp.dot(h, wa1, preferred_element_type=F32) + ba1)
        return jnp.sum(e * wa2, axis=1, keepdims=True)

    ef = score(hf)
    et = score(ht)
    ei = score(hi)
    mx = jnp.maximum(jnp.maximum(ef, et), ei)
    xf = jnp.exp(ef - mx)
    xt = jnp.exp(et - mx)
    xi = jnp.exp(ei - mx)
    s = xf + xt + xi
    bf = xf / s
    bt = xt / s
    bi = xi / s
    bf_ref[...] = bf
    bt_ref[...] = bt
    bi_ref[...] = bi
    hid_ref[...] = bf * hf + bt * ht + bi * hi


def _wadj_body(nsteps, f_ref, t_ref, i_ref, bf_ref, bt_ref, bi_ref,
               a_ref, cs_ref, rs_ref):
    k = pl.program_id(0)
    blk = (bf_ref[...] * f_ref[...].astype(F32)
           + bt_ref[...] * t_ref[...].astype(F32)
           + bi_ref[...] * i_ref[...].astype(F32))
    a_ref[...] = blk.astype(BF16)
    rs_ref[...] = jnp.sum(blk, axis=1, keepdims=True)

    @pl.when(k == 0)
    def _():
        cs_ref[...] = jnp.zeros_like(cs_ref)

    cs_ref[...] += jnp.sum(blk, axis=0, keepdims=True)


def _weighted_adj(adj_f, adj_t, adj_i, bf, bt, bi, bk=256):
    n = adj_f.shape[0]
    nk = n // bk
    return pl.pallas_call(
        functools.partial(_wadj_body, nk),
        grid=(nk,),
        in_specs=[pl.BlockSpec((bk, n), lambda k: (k, 0))] * 3
        + [pl.BlockSpec((bk, 1), lambda k: (k, 0))] * 3,
        out_specs=[
            pl.BlockSpec((bk, n), lambda k: (k, 0)),
            pl.BlockSpec((1, n), lambda k: (0, 0)),
            pl.BlockSpec((bk, 1), lambda k: (k, 0)),
        ],
        out_shape=[
            jax.ShapeDtypeStruct((n, n), BF16),
            jax.ShapeDtypeStruct((1, n), F32),
            jax.ShapeDtypeStruct((n, 1), F32),
        ],
    )(adj_f, adj_t, adj_i, bf, bt, bi)


def _dec_in_body(cs_ref, rs_ref, hid_ref, w3_ref, y3_ref, dis_ref):
    dw = (cs_ref[...] + rs_ref[...]) / 3.0
    dis = jnp.where(dw > 0, jax.lax.rsqrt(dw), 0.0)
    dis_ref[...] = dis
    y3_ref[...] = (dis * jnp.dot(hid_ref[...], w3_ref[...],
                                 preferred_element_type=F32)).astype(BF16)


def _sym_body(nk, relu, a1_ref, a2_ref, y_ref, dis_ref, b_ref, o_ref):
    k = pl.program_id(1)
    y = y_ref[...]
    acc = jax.lax.dot_general(
        a1_ref[...], y, (((1,), (0,)), ((), ())), preferred_element_type=F32)
    acc += jax.lax.dot_general(
        a2_ref[...], y, (((0,), (0,)), ((), ())), preferred_element_type=F32)

    @pl.when(k == 0)
    def _():
        o_ref[...] = acc

    @pl.when(k > 0)
    def _():
        o_ref[...] += acc

    @pl.when(k == nk - 1)
    def _():
        r = dis_ref[...] * (o_ref[...] * (1.0 / 3.0)) + b_ref[...]
        if relu:
            r = jnp.maximum(r, 0.0)
        o_ref[...] = r


def _sym_mm(adj123, Y, dis_col, bias_row, relu, bi=512, bk=512):
    n = adj123.shape[0]
    w = Y.shape[1]
    gi, gk = n // bi, n // bk
    return pl.pallas_call(
        functools.partial(_sym_body, gk, relu),
        grid=(gi, gk),
        in_specs=[
            pl.BlockSpec((bi, bk), lambda i, k: (i, k)),
            pl.BlockSpec((bk, bi), lambda i, k: (k, i)),
            pl.BlockSpec((bk, w), lambda i, k: (k, 0)),
            pl.BlockSpec((bi, 1), lambda i, k: (i, 0)),
            pl.BlockSpec((1, w), lambda i, k: (0, 0)),
        ],
        out_specs=pl.BlockSpec((bi, w), lambda i, k: (i, 0)),
        out_shape=jax.ShapeDtypeStruct((n, w), F32),
    )(adj123, adj123, Y, dis_col, bias_row)


def _dec_mid_body(h_ref, g_ref, b_ref, w4_ref, dis_ref, y4_ref):
    y4_ref[...] = (dis_ref[...] * jnp.dot(
        _bn_relu(h_ref[...], g_ref[...], b_ref[...]), w4_ref[...],
        preferred_element_type=F32)).astype(BF16)


def _simple_call(body, out_shapes, *args):
    return pl.pallas_call(body, out_shape=out_shapes)(*args)


def kernel(feat, feat_a, adj_f, adj_t, adj_i, W1, b1, W2, b2, W3, b3, W4, b4,
           bn_gamma, bn_beta, Wa1, ba1, Wa2, Wb, bb):
    n = adj_f.shape[0]
    din = W1.shape[0]
    dh = W1.shape[1]
    dout = W2.shape[1]

    gamma = bn_gamma.reshape(1, dh)
    beta = bn_beta.reshape(1, dh)
    b1r = b1.reshape(1, dh)
    b2r = b2.reshape(1, dout)
    b3r = b3.reshape(1, dh)
    b4r = b4.reshape(1, din)
    ba1r = ba1.reshape(1, -1)
    wa2r = Wa2.reshape(1, -1)
    bbr = bb.reshape(1, 1)
    wb0 = Wb[0]

    dis_f, dis_t, dis_i, rs_t, adj_fb, adj_tb, adj_ib = _degrees(
        adj_f, adj_t, adj_i)
    dis_f = dis_f.reshape(n, 1)
    dis_t = dis_t.reshape(n, 1)
    dis_i = dis_i.reshape(n, 1)

    sds = jax.ShapeDtypeStruct((n, dh), BF16)
    y1_f, y1_t, y1_a, y1_i = _simple_call(
        _prep1_body, [sds, sds, sds, sds],
        feat, feat_a, W1, dis_f, dis_t, dis_i)
    y1_ta = jnp.concatenate([y1_t, y1_a], axis=1)

    h1a_f = _gcn_mmT(adj_fb, y1_f, dis_f, b1r)
    h1a_ta = _gcn_mmT(adj_tb, y1_ta, dis_t, jnp.concatenate([b1r, b1r], axis=1))
    h1a_i = _gcn_mmT(adj_ib, y1_i, dis_i, b1r)

    sds2 = jax.ShapeDtypeStruct((n, dout), BF16)
    y2_f, y2_t, y2_a, y2_i = _simple_call(
        _mid_body, [sds2, sds2, sds2, sds2],
        h1a_f, h1a_ta, h1a_i, gamma, beta, W2, dis_f, dis_t, dis_i)
    y2_ta = jnp.concatenate([y2_t, y2_a], axis=1)

    h1_f = _gcn_mmT(adj_fb, y2_f, dis_f, b2r)
    h1_ta = _gcn_mmT(adj_tb, y2_ta, dis_t, jnp.concatenate([b2r, b2r], axis=1))
    h1_i = _gcn_mmT(adj_ib, y2_i, dis_i, b2r)
    h1_t = h1_ta[:, :dout]
    h1_a = h1_ta[:, dout:]

    g = _readout(adj_tb, h1_t, rs_t)

    col = jax.ShapeDtypeStruct((n, 1), F32)
    sc1, sc2, hiden_emb, att_f, att_t, att_i = _simple_call(
        _att_body,
        [col, col, jax.ShapeDtypeStruct((n, dout), F32), col, col, col],
        h1_f, h1_t, h1_a, h1_i, g, wb0, bbr, Wa1, ba1r, wa2r)
    ret = jnp.concatenate([sc1, sc2], axis=1)

    adj123, cs123, rs123 = _weighted_adj(adj_fb, adj_tb, adj_ib,
                                         att_f, att_t, att_i)

    y3, dis_w = _simple_call(
        _dec_in_body,
        [jax.ShapeDtypeStruct((n, dh), BF16),
         jax.ShapeDtypeStruct((n, 1), F32)],
        cs123.reshape(n, 1), rs123, hiden_emb, W3)

    h2a = _sym_mm(adj123, y3, dis_w, b3r, relu=False)

    y4 = _simple_call(
        _dec_mid_body, jax.ShapeDtypeStruct((n, din), BF16),
        h2a, gamma, beta, W4, dis_w)

    h2 = _sym_mm(adj123, y4, dis_w, b4r, relu=True)

    return (hiden_emb, h2, ret)


# speedup vs baseline: 1.7830x; 1.5429x over previous
import functools

import jax
import jax.numpy as jnp
from jax.experimental import pallas as pl

F32 = jnp.float32
BF16 = jnp.bfloat16


def _deg_body(nsteps, f_ref, t_ref, i_ref,
              df_ref, dt_ref, di_ref, rs_ref, fb_ref, tb_ref, ib_ref):
    k = pl.program_id(0)
    f = f_ref[...]
    t = t_ref[...]
    i = i_ref[...]
    fb_ref[...] = f.astype(BF16)
    tb_ref[...] = t.astype(BF16)
    ib_ref[...] = i.astype(BF16)
    rs_ref[...] = jnp.sum(t, axis=1, keepdims=True)

    @pl.when(k == 0)
    def _():
        df_ref[...] = jnp.zeros_like(df_ref)
        dt_ref[...] = jnp.zeros_like(dt_ref)
        di_ref[...] = jnp.zeros_like(di_ref)

    df_ref[...] += jnp.sum(f, axis=0, keepdims=True)
    dt_ref[...] += jnp.sum(t, axis=0, keepdims=True)
    di_ref[...] += jnp.sum(i, axis=0, keepdims=True)

    @pl.when(k == nsteps - 1)
    def _():
        for r in (df_ref, dt_ref, di_ref):
            s = r[...]
            r[...] = jnp.where(s > 0, jax.lax.rsqrt(s), 0.0)


def _degrees(adj_f, adj_t, adj_i, bk=256):
    n = adj_f.shape[0]
    nk = n // bk
    return pl.pallas_call(
        functools.partial(_deg_body, nk),
        grid=(nk,),
        in_specs=[pl.BlockSpec((bk, n), lambda k: (k, 0))] * 3,
        out_specs=[pl.BlockSpec((1, n), lambda k: (0, 0))] * 3
        + [pl.BlockSpec((bk, 1), lambda k: (k, 0))]
        + [pl.BlockSpec((bk, n), lambda k: (k, 0))] * 3,
        out_shape=[jax.ShapeDtypeStruct((1, n), F32)] * 3
        + [jax.ShapeDtypeStruct((n, 1), F32)]
        + [jax.ShapeDtypeStruct((n, n), BF16)] * 3,
    )(adj_f, adj_t, adj_i)


def _mmT_body(nk, a_ref, y_ref, dis_ref, b_ref, o_ref):
    k = pl.program_id(1)
    acc = jax.lax.dot_general(
        a_ref[...], y_ref[...], (((0,), (0,)), ((), ())),
        preferred_element_type=F32)

    @pl.when(k == 0)
    def _():
        o_ref[...] = acc

    @pl.when(k > 0)
    def _():
        o_ref[...] += acc

    @pl.when(k == nk - 1)
    def _():
        o_ref[...] = o_ref[...] * dis_ref[...] + b_ref[...]


def _gcn_mmT(A, Y, dis_col, bias_row, bi=1024, bk=1024):
    n = A.shape[0]
    w = Y.shape[1]
    gi, gk = n // bi, n // bk
    return pl.pallas_call(
        functools.partial(_mmT_body, gk),
        grid=(gi, gk),
        in_specs=[
            pl.BlockSpec((bk, bi), lambda i, k: (k, i)),
            pl.BlockSpec((bk, w), lambda i, k: (k, 0)),
            pl.BlockSpec((bi, 1), lambda i, k: (i, 0)),
            pl.BlockSpec((1, w), lambda i, k: (0, 0)),
        ],
        out_specs=pl.BlockSpec((bi, w), lambda i, k: (i, 0)),
        out_shape=jax.ShapeDtypeStruct((n, w), F32),
    )(A, Y, dis_col, bias_row)


def _prep1_body(feat_ref, feata_ref, w1_ref, df_ref, dt_ref, di_ref,
                yf_ref, yt_ref, ya_ref, yi_ref):
    w1 = w1_ref[...]
    p = jnp.dot(feat_ref[...], w1, preferred_element_type=F32)
    pa = jnp.dot(feata_ref[...], w1, preferred_element_type=F32)
    yf_ref[...] = (df_ref[...] * p).astype(BF16)
    yt_ref[...] = (dt_ref[...] * p).astype(BF16)
    ya_ref[...] = (dt_ref[...] * pa).astype(BF16)
    yi_ref[...] = (di_ref[...] * p).astype(BF16)


def _bn_relu(h, gamma, beta):
    m = jnp.mean(h, axis=0, keepdims=True)
    c = h - m
    v = jnp.mean(c * c, axis=0, keepdims=True)
    return jnp.maximum(c * jax.lax.rsqrt(v + 1e-5) * gamma + beta, 0.0)


def _mid_body(hf_ref, hta_ref, hi_ref, g_ref, b_ref, w2_ref,
              df_ref, dt_ref, di_ref, yf_ref, yt_ref, ya_ref, yi_ref):
    g = g_ref[...]
    b = b_ref[...]
    w2 = w2_ref[...]
    dh = w2.shape[0]

    def stage(h, dis):
        return (dis * jnp.dot(_bn_relu(h, g, b), w2,
                              preferred_element_type=F32)).astype(BF16)

    yf_ref[...] = stage(hf_ref[...], df_ref[...])
    yt_ref[...] = stage(hta_ref[:, :dh], dt_ref[...])
    ya_ref[...] = stage(hta_ref[:, dh:], dt_ref[...])
    yi_ref[...] = stage(hi_ref[...], di_ref[...])


def _readout_body(nk, a_ref, y_ref, rs_ref, o_ref):
    k = pl.program_id(1)
    acc = jnp.dot(a_ref[...], y_ref[...].astype(BF16),
                  preferred_element_type=F32)

    @pl.when(k == 0)
    def _():
        o_ref[...] = acc

    @pl.when(k > 0)
    def _():
        o_ref[...] += acc

    @pl.when(k == nk - 1)
    def _():
        gg = o_ref[...] / rs_ref[...]
        nrm = jnp.sqrt(jnp.sum(gg * gg, axis=1, keepdims=True))
        gg = gg / jnp.maximum(nrm, 1e-12)
        o_ref[...] = jax.nn.sigmoid(gg)


def _readout(adj_t, h1_t, rs_t, bi=1024, bk=1024):
    n = adj_t.shape[0]
    w = h1_t.shape[1]
    gi, gk = n // bi, n // bk
    return pl.pallas_call(
        functools.partial(_readout_body, gk),
        grid=(gi, gk),
        in_specs=[
            pl.BlockSpec((bi, bk), lambda i, k: (i, k)),
            pl.BlockSpec((bk, w), lambda i, k: (k, 0)),
            pl.BlockSpec((bi, 1), lambda i, k: (i, 0)),
        ],
        out_specs=pl.BlockSpec((bi, w), lambda i, k: (i, 0)),
        out_shape=jax.ShapeDtypeStruct((n, w), F32),
    )(adj_t, h1_t, rs_t)


def _att_body(hf_ref, ht_ref, ha_ref, hi_ref, g_ref, wb_ref, bb_ref,
              wa1_ref, ba1_ref, wa2_ref,
              sc1_ref, sc2_ref, hid_ref, bf_ref, bt_ref, bi_ref):
    hf = hf_ref[...]
    ht = ht_ref[...]
    ha = ha_ref[...]
    hi = hi_ref[...]
    g = g_ref[...]
    t = jax.lax.dot_general(
        g, wb_ref[...], (((1,), (1,)), ((), ())), preferred_element_type=F32)
    sc1_ref[...] = jnp.sum(ht * t, axis=1, keepdims=True) + bb_ref[...]
    sc2_ref[...] = jnp.sum(ha * t, axis=1, keepdims=True) + bb_ref[...]

    wa1 = wa1_ref[...]
    ba1 = ba1_ref[...]
    wa2 = wa2_ref[...]

    def score(h):
        e = jnp.tanh(jnp.dot(h, wa1, preferred_element_type=F32) + ba1)
        return jnp.sum(e * wa2, axis=1, keepdims=True)

    ef = score(hf)
    et = score(ht)
    ei = score(hi)
    mx = jnp.maximum(jnp.maximum(ef, et), ei)
    xf = jnp.exp(ef - mx)
    xt = jnp.exp(et - mx)
    xi = jnp.exp(ei - mx)
    s = xf + xt + xi
    bf = xf / s
    bt = xt / s
    bi = xi / s
    bf_ref[...] = bf
    bt_ref[...] = bt
    bi_ref[...] = bi
    hid_ref[...] = bf * hf + bt * ht + bi * hi


def _wadj_body(nsteps, f_ref, t_ref, i_ref, bf_ref, bt_ref, bi_ref,
               a_ref, cs_ref, rs_ref):
    k = pl.program_id(0)
    blk = (bf_ref[...] * f_ref[...].astype(F32)
           + bt_ref[...] * t_ref[...].astype(F32)
           + bi_ref[...] * i_ref[...].astype(F32))
    a_ref[...] = blk.astype(BF16)
    rs_ref[...] = jnp.sum(blk, axis=1, keepdims=True)

    @pl.when(k == 0)
    def _():
        cs_ref[...] = jnp.zeros_like(cs_ref)

    cs_ref[...] += jnp.sum(blk, axis=0, keepdims=True)


def _weighted_adj(adj_f, adj_t, adj_i, bf, bt, bi, bk=256):
    n = adj_f.shape[0]
    nk = n // bk
    return pl.pallas_call(
        functools.partial(_wadj_body, nk),
        grid=(nk,),
        in_specs=[pl.BlockSpec((bk, n), lambda k: (k, 0))] * 3
        + [pl.BlockSpec((bk, 1), lambda k: (k, 0))] * 3,
        out_specs=[
            pl.BlockSpec((bk, n), lambda k: (k, 0)),
            pl.BlockSpec((1, n), lambda k: (0, 0)),
            pl.BlockSpec((bk, 1), lambda k: (k, 0)),
        ],
        out_shape=[
            jax.ShapeDtypeStruct((n, n), BF16),
            jax.ShapeDtypeStruct((1, n), F32),
            jax.ShapeDtypeStruct((n, 1), F32),
        ],
    )(adj_f, adj_t, adj_i, bf, bt, bi)


def _dec_in_body(cs_ref, rs_ref, hid_ref, w3_ref, y3_ref, dis_ref):
    dw = (cs_ref[...] + rs_ref[...]) / 3.0
    dis = jnp.where(dw > 0, jax.lax.rsqrt(dw), 0.0)
    dis_ref[...] = dis
    y3_ref[...] = (dis * jnp.dot(hid_ref[...], w3_ref[...],
                                 preferred_element_type=F32)).astype(BF16)


def _sym_body(nk, relu, a1_ref, a2_ref, y_ref, dis_ref, b_ref, o_ref):
    k = pl.program_id(1)
    y = y_ref[...]
    acc = jax.lax.dot_general(
        a1_ref[...], y, (((1,), (0,)), ((), ())), preferred_element_type=F32)
    acc += jax.lax.dot_general(
        a2_ref[...], y, (((0,), (0,)), ((), ())), preferred_element_type=F32)

    @pl.when(k == 0)
    def _():
        o_ref[...] = acc

    @pl.when(k > 0)
    def _():
        o_ref[...] += acc

    @pl.when(k == nk - 1)
    def _():
        r = dis_ref[...] * (o_ref[...] * (1.0 / 3.0)) + b_ref[...]
        if relu:
            r = jnp.maximum(r, 0.0)
        o_ref[...] = r


def _sym_mm(adj123, Y, dis_col, bias_row, relu, bi=1024, bk=1024):
    n = adj123.shape[0]
    w = Y.shape[1]
    gi, gk = n // bi, n // bk
    return pl.pallas_call(
        functools.partial(_sym_body, gk, relu),
        grid=(gi, gk),
        in_specs=[
            pl.BlockSpec((bi, bk), lambda i, k: (i, k)),
            pl.BlockSpec((bk, bi), lambda i, k: (k, i)),
            pl.BlockSpec((bk, w), lambda i, k: (k, 0)),
            pl.BlockSpec((bi, 1), lambda i, k: (i, 0)),
            pl.BlockSpec((1, w), lambda i, k: (0, 0)),
        ],
        out_specs=pl.BlockSpec((bi, w), lambda i, k: (i, 0)),
        out_shape=jax.ShapeDtypeStruct((n, w), F32),
    )(adj123, adj123, Y, dis_col, bias_row)


def _dec_mid_body(h_ref, g_ref, b_ref, w4_ref, dis_ref, y4_ref):
    y4_ref[...] = (dis_ref[...] * jnp.dot(
        _bn_relu(h_ref[...], g_ref[...], b_ref[...]), w4_ref[...],
        preferred_element_type=F32)).astype(BF16)


def _simple_call(body, out_shapes, *args):
    return pl.pallas_call(body, out_shape=out_shapes)(*args)


def kernel(feat, feat_a, adj_f, adj_t, adj_i, W1, b1, W2, b2, W3, b3, W4, b4,
           bn_gamma, bn_beta, Wa1, ba1, Wa2, Wb, bb):
    n = adj_f.shape[0]
    din = W1.shape[0]
    dh = W1.shape[1]
    dout = W2.shape[1]

    gamma = bn_gamma.reshape(1, dh)
    beta = bn_beta.reshape(1, dh)
    b1r = b1.reshape(1, dh)
    b2r = b2.reshape(1, dout)
    b3r = b3.reshape(1, dh)
    b4r = b4.reshape(1, din)
    ba1r = ba1.reshape(1, -1)
    wa2r = Wa2.reshape(1, -1)
    bbr = bb.reshape(1, 1)
    wb0 = Wb[0]

    dis_f, dis_t, dis_i, rs_t, adj_fb, adj_tb, adj_ib = _degrees(
        adj_f, adj_t, adj_i)
    dis_f = dis_f.reshape(n, 1)
    dis_t = dis_t.reshape(n, 1)
    dis_i = dis_i.reshape(n, 1)

    sds = jax.ShapeDtypeStruct((n, dh), BF16)
    y1_f, y1_t, y1_a, y1_i = _simple_call(
        _prep1_body, [sds, sds, sds, sds],
        feat, feat_a, W1, dis_f, dis_t, dis_i)
    y1_ta = jnp.concatenate([y1_t, y1_a], axis=1)

    h1a_f = _gcn_mmT(adj_fb, y1_f, dis_f, b1r)
    h1a_ta = _gcn_mmT(adj_tb, y1_ta, dis_t, jnp.concatenate([b1r, b1r], axis=1))
    h1a_i = _gcn_mmT(adj_ib, y1_i, dis_i, b1r)

    sds2 = jax.ShapeDtypeStruct((n, dout), BF16)
    y2_f, y2_t, y2_a, y2_i = _simple_call(
        _mid_body, [sds2, sds2, sds2, sds2],
        h1a_f, h1a_ta, h1a_i, gamma, beta, W2, dis_f, dis_t, dis_i)
    y2_ta = jnp.concatenate([y2_t, y2_a], axis=1)

    h1_f = _gcn_mmT(adj_fb, y2_f, dis_f, b2r)
    h1_ta = _gcn_mmT(adj_tb, y2_ta, dis_t, jnp.concatenate([b2r, b2r], axis=1))
    h1_i = _gcn_mmT(adj_ib, y2_i, dis_i, b2r)
    h1_t = h1_ta[:, :dout]
    h1_a = h1_ta[:, dout:]

    g = _readout(adj_tb, h1_t, rs_t)

    col = jax.ShapeDtypeStruct((n, 1), F32)
    sc1, sc2, hiden_emb, att_f, att_t, att_i = _simple_call(
        _att_body,
        [col, col, jax.ShapeDtypeStruct((n, dout), F32), col, col, col],
        h1_f, h1_t, h1_a, h1_i, g, wb0, bbr, Wa1, ba1r, wa2r)
    ret = jnp.concatenate([sc1, sc2], axis=1)

    adj123, cs123, rs123 = _weighted_adj(adj_fb, adj_tb, adj_ib,
                                         att_f, att_t, att_i)

    y3, dis_w = _simple_call(
        _dec_in_body,
        [jax.ShapeDtypeStruct((n, dh), BF16),
         jax.ShapeDtypeStruct((n, 1), F32)],
        cs123.reshape(n, 1), rs123, hiden_emb, W3)

    h2a = _sym_mm(adj123, y3, dis_w, b3r, relu=False)

    y4 = _simple_call(
        _dec_mid_body, jax.ShapeDtypeStruct((n, din), BF16),
        h2a, gamma, beta, W4, dis_w)

    h2 = _sym_mm(adj123, y4, dis_w, b4r, relu=True)

    return (hiden_emb, h2, ret)


# speedup vs baseline: 2.1206x; 1.1894x over previous
import functools

import jax
import jax.numpy as jnp
from jax.experimental import pallas as pl

F32 = jnp.float32
BF16 = jnp.bfloat16


def _deg_body(nsteps, f_ref, t_ref, i_ref,
              df_ref, dt_ref, di_ref, rs_ref, fb_ref, tb_ref, ib_ref):
    k = pl.program_id(0)
    f = f_ref[...]
    t = t_ref[...]
    i = i_ref[...]
    fb_ref[...] = f.astype(BF16)
    tb_ref[...] = t.astype(BF16)
    ib_ref[...] = i.astype(BF16)
    rs_ref[...] = jnp.sum(t, axis=1, keepdims=True)

    @pl.when(k == 0)
    def _():
        df_ref[...] = jnp.zeros_like(df_ref)
        dt_ref[...] = jnp.zeros_like(dt_ref)
        di_ref[...] = jnp.zeros_like(di_ref)

    df_ref[...] += jnp.sum(f, axis=0, keepdims=True)
    dt_ref[...] += jnp.sum(t, axis=0, keepdims=True)
    di_ref[...] += jnp.sum(i, axis=0, keepdims=True)

    @pl.when(k == nsteps - 1)
    def _():
        for r in (df_ref, dt_ref, di_ref):
            s = r[...]
            r[...] = jnp.where(s > 0, jax.lax.rsqrt(s), 0.0)


def _degrees(adj_f, adj_t, adj_i, bk=256):
    n = adj_f.shape[0]
    nk = n // bk
    return pl.pallas_call(
        functools.partial(_deg_body, nk),
        grid=(nk,),
        in_specs=[pl.BlockSpec((bk, n), lambda k: (k, 0))] * 3,
        out_specs=[pl.BlockSpec((1, n), lambda k: (0, 0))] * 3
        + [pl.BlockSpec((bk, 1), lambda k: (k, 0))]
        + [pl.BlockSpec((bk, n), lambda k: (k, 0))] * 3,
        out_shape=[jax.ShapeDtypeStruct((1, n), F32)] * 3
        + [jax.ShapeDtypeStruct((n, 1), F32)]
        + [jax.ShapeDtypeStruct((n, n), BF16)] * 3,
    )(adj_f, adj_t, adj_i)


def _mmT_body(nk, a_ref, y_ref, dis_ref, b_ref, o_ref):
    k = pl.program_id(1)
    acc = jax.lax.dot_general(
        a_ref[...], y_ref[...], (((0,), (0,)), ((), ())),
        preferred_element_type=F32)

    @pl.when(k == 0)
    def _():
        o_ref[...] = acc

    @pl.when(k > 0)
    def _():
        o_ref[...] += acc

    @pl.when(k == nk - 1)
    def _():
        o_ref[...] = o_ref[...] * dis_ref[...] + b_ref[...]


def _gcn_mmT(A, Y, dis_col, bias_row, bi=1024, bk=4096):
    n = A.shape[0]
    w = Y.shape[1]
    gi, gk = n // bi, n // bk
    return pl.pallas_call(
        functools.partial(_mmT_body, gk),
        grid=(gi, gk),
        in_specs=[
            pl.BlockSpec((bk, bi), lambda i, k: (k, i)),
            pl.BlockSpec((bk, w), lambda i, k: (k, 0)),
            pl.BlockSpec((bi, 1), lambda i, k: (i, 0)),
            pl.BlockSpec((1, w), lambda i, k: (0, 0)),
        ],
        out_specs=pl.BlockSpec((bi, w), lambda i, k: (i, 0)),
        out_shape=jax.ShapeDtypeStruct((n, w), F32),
    )(A, Y, dis_col, bias_row)


def _prep1_body(feat_ref, feata_ref, w1_ref, df_ref, dt_ref, di_ref,
                yf_ref, yt_ref, ya_ref, yi_ref):
    w1 = w1_ref[...]
    p = jnp.dot(feat_ref[...], w1, preferred_element_type=F32)
    pa = jnp.dot(feata_ref[...], w1, preferred_element_type=F32)
    yf_ref[...] = (df_ref[...] * p).astype(BF16)
    yt_ref[...] = (dt_ref[...] * p).astype(BF16)
    ya_ref[...] = (dt_ref[...] * pa).astype(BF16)
    yi_ref[...] = (di_ref[...] * p).astype(BF16)


def _bn_relu(h, gamma, beta):
    m = jnp.mean(h, axis=0, keepdims=True)
    c = h - m
    v = jnp.mean(c * c, axis=0, keepdims=True)
    return jnp.maximum(c * jax.lax.rsqrt(v + 1e-5) * gamma + beta, 0.0)


def _mid_body(hf_ref, hta_ref, hi_ref, g_ref, b_ref, w2_ref,
              df_ref, dt_ref, di_ref, yf_ref, yt_ref, ya_ref, yi_ref):
    g = g_ref[...]
    b = b_ref[...]
    w2 = w2_ref[...]
    dh = w2.shape[0]

    def stage(h, dis):
        return (dis * jnp.dot(_bn_relu(h, g, b), w2,
                              preferred_element_type=F32)).astype(BF16)

    yf_ref[...] = stage(hf_ref[...], df_ref[...])
    yt_ref[...] = stage(hta_ref[:, :dh], dt_ref[...])
    ya_ref[...] = stage(hta_ref[:, dh:], dt_ref[...])
    yi_ref[...] = stage(hi_ref[...], di_ref[...])


def _readout_body(nk, a_ref, y_ref, rs_ref, o_ref):
    k = pl.program_id(1)
    acc = jnp.dot(a_ref[...], y_ref[...].astype(BF16),
                  preferred_element_type=F32)

    @pl.when(k == 0)
    def _():
        o_ref[...] = acc

    @pl.when(k > 0)
    def _():
        o_ref[...] += acc

    @pl.when(k == nk - 1)
    def _():
        gg = o_ref[...] / rs_ref[...]
        nrm = jnp.sqrt(jnp.sum(gg * gg, axis=1, keepdims=True))
        gg = gg / jnp.maximum(nrm, 1e-12)
        o_ref[...] = jax.nn.sigmoid(gg)


def _readout(adj_t, h1_t, rs_t, bi=1024, bk=4096):
    n = adj_t.shape[0]
    w = h1_t.shape[1]
    gi, gk = n // bi, n // bk
    return pl.pallas_call(
        functools.partial(_readout_body, gk),
        grid=(gi, gk),
        in_specs=[
            pl.BlockSpec((bi, bk), lambda i, k: (i, k)),
            pl.BlockSpec((bk, w), lambda i, k: (k, 0)),
            pl.BlockSpec((bi, 1), lambda i, k: (i, 0)),
        ],
        out_specs=pl.BlockSpec((bi, w), lambda i, k: (i, 0)),
        out_shape=jax.ShapeDtypeStruct((n, w), F32),
    )(adj_t, h1_t, rs_t)


def _att_body(hf_ref, ht_ref, ha_ref, hi_ref, g_ref, wb_ref, bb_ref,
              wa1_ref, ba1_ref, wa2_ref,
              sc1_ref, sc2_ref, hid_ref, bf_ref, bt_ref, bi_ref):
    hf = hf_ref[...]
    ht = ht_ref[...]
    ha = ha_ref[...]
    hi = hi_ref[...]
    g = g_ref[...]
    t = jax.lax.dot_general(
        g, wb_ref[...], (((1,), (1,)), ((), ())), preferred_element_type=F32)
    sc1_ref[...] = jnp.sum(ht * t, axis=1, keepdims=True) + bb_ref[...]
    sc2_ref[...] = jnp.sum(ha * t, axis=1, keepdims=True) + bb_ref[...]

    wa1 = wa1_ref[...]
    ba1 = ba1_ref[...]
    wa2 = wa2_ref[...]

    def score(h):
        e = jnp.tanh(jnp.dot(h, wa1, preferred_element_type=F32) + ba1)
        return jnp.sum(e * wa2, axis=1, keepdims=True)

    ef = score(hf)
    et = score(ht)
    ei = score(hi)
    mx = jnp.maximum(jnp.maximum(ef, et), ei)
    xf = jnp.exp(ef - mx)
    xt = jnp.exp(et - mx)
    xi = jnp.exp(ei - mx)
    s = xf + xt + xi
    bf = xf / s
    bt = xt / s
    bi = xi / s
    bf_ref[...] = bf
    bt_ref[...] = bt
    bi_ref[...] = bi
    hid_ref[...] = bf * hf + bt * ht + bi * hi


def _wadj_body(nsteps, f_ref, t_ref, i_ref, bf_ref, bt_ref, bi_ref,
               a_ref, cs_ref, rs_ref):
    k = pl.program_id(0)
    blk = (bf_ref[...] * f_ref[...].astype(F32)
           + bt_ref[...] * t_ref[...].astype(F32)
           + bi_ref[...] * i_ref[...].astype(F32))
    a_ref[...] = blk.astype(BF16)
    rs_ref[...] = jnp.sum(blk, axis=1, keepdims=True)

    @pl.when(k == 0)
    def _():
        cs_ref[...] = jnp.zeros_like(cs_ref)

    cs_ref[...] += jnp.sum(blk, axis=0, keepdims=True)


def _weighted_adj(adj_f, adj_t, adj_i, bf, bt, bi, bk=256):
    n = adj_f.shape[0]
    nk = n // bk
    return pl.pallas_call(
        functools.partial(_wadj_body, nk),
        grid=(nk,),
        in_specs=[pl.BlockSpec((bk, n), lambda k: (k, 0))] * 3
        + [pl.BlockSpec((bk, 1), lambda k: (k, 0))] * 3,
        out_specs=[
            pl.BlockSpec((bk, n), lambda k: (k, 0)),
            pl.BlockSpec((1, n), lambda k: (0, 0)),
            pl.BlockSpec((bk, 1), lambda k: (k, 0)),
        ],
        out_shape=[
            jax.ShapeDtypeStruct((n, n), BF16),
            jax.ShapeDtypeStruct((1, n), F32),
            jax.ShapeDtypeStruct((n, 1), F32),
        ],
    )(adj_f, adj_t, adj_i, bf, bt, bi)


def _dec_in_body(cs_ref, rs_ref, hid_ref, w3_ref, y3_ref, dis_ref):
    dw = (cs_ref[...] + rs_ref[...]) / 3.0
    dis = jnp.where(dw > 0, jax.lax.rsqrt(dw), 0.0)
    dis_ref[...] = dis
    y3_ref[...] = (dis * jnp.dot(hid_ref[...], w3_ref[...],
                                 preferred_element_type=F32)).astype(BF16)


def _sym_body(nk, relu, a1_ref, a2_ref, y_ref, dis_ref, b_ref, o_ref):
    k = pl.program_id(1)
    y = y_ref[...]
    acc = jax.lax.dot_general(
        a1_ref[...], y, (((1,), (0,)), ((), ())), preferred_element_type=F32)
    acc += jax.lax.dot_general(
        a2_ref[...], y, (((0,), (0,)), ((), ())), preferred_element_type=F32)

    @pl.when(k == 0)
    def _():
        o_ref[...] = acc

    @pl.when(k > 0)
    def _():
        o_ref[...] += acc

    @pl.when(k == nk - 1)
    def _():
        r = dis_ref[...] * (o_ref[...] * (1.0 / 3.0)) + b_ref[...]
        if relu:
            r = jnp.maximum(r, 0.0)
        o_ref[...] = r


def _sym_mm(adj123, Y, dis_col, bias_row, relu, bi=1024, bk=4096):
    n = adj123.shape[0]
    w = Y.shape[1]
    gi, gk = n // bi, n // bk
    return pl.pallas_call(
        functools.partial(_sym_body, gk, relu),
        grid=(gi, gk),
        in_specs=[
            pl.BlockSpec((bi, bk), lambda i, k: (i, k)),
            pl.BlockSpec((bk, bi), lambda i, k: (k, i)),
            pl.BlockSpec((bk, w), lambda i, k: (k, 0)),
            pl.BlockSpec((bi, 1), lambda i, k: (i, 0)),
            pl.BlockSpec((1, w), lambda i, k: (0, 0)),
        ],
        out_specs=pl.BlockSpec((bi, w), lambda i, k: (i, 0)),
        out_shape=jax.ShapeDtypeStruct((n, w), F32),
    )(adj123, adj123, Y, dis_col, bias_row)


def _dec_mid_body(h_ref, g_ref, b_ref, w4_ref, dis_ref, y4_ref):
    y4_ref[...] = (dis_ref[...] * jnp.dot(
        _bn_relu(h_ref[...], g_ref[...], b_ref[...]), w4_ref[...],
        preferred_element_type=F32)).astype(BF16)


def _simple_call(body, out_shapes, *args):
    return pl.pallas_call(body, out_shape=out_shapes)(*args)


def kernel(feat, feat_a, adj_f, adj_t, adj_i, W1, b1, W2, b2, W3, b3, W4, b4,
           bn_gamma, bn_beta, Wa1, ba1, Wa2, Wb, bb):
    n = adj_f.shape[0]
    din = W1.shape[0]
    dh = W1.shape[1]
    dout = W2.shape[1]

    gamma = bn_gamma.reshape(1, dh)
    beta = bn_beta.reshape(1, dh)
    b1r = b1.reshape(1, dh)
    b2r = b2.reshape(1, dout)
    b3r = b3.reshape(1, dh)
    b4r = b4.reshape(1, din)
    ba1r = ba1.reshape(1, -1)
    wa2r = Wa2.reshape(1, -1)
    bbr = bb.reshape(1, 1)
    wb0 = Wb[0]

    dis_f, dis_t, dis_i, rs_t, adj_fb, adj_tb, adj_ib = _degrees(
        adj_f, adj_t, adj_i)
    dis_f = dis_f.reshape(n, 1)
    dis_t = dis_t.reshape(n, 1)
    dis_i = dis_i.reshape(n, 1)

    sds = jax.ShapeDtypeStruct((n, dh), BF16)
    y1_f, y1_t, y1_a, y1_i = _simple_call(
        _prep1_body, [sds, sds, sds, sds],
        feat, feat_a, W1, dis_f, dis_t, dis_i)
    y1_ta = jnp.concatenate([y1_t, y1_a], axis=1)

    h1a_f = _gcn_mmT(adj_fb, y1_f, dis_f, b1r)
    h1a_ta = _gcn_mmT(adj_tb, y1_ta, dis_t, jnp.concatenate([b1r, b1r], axis=1))
    h1a_i = _gcn_mmT(adj_ib, y1_i, dis_i, b1r)

    sds2 = jax.ShapeDtypeStruct((n, dout), BF16)
    y2_f, y2_t, y2_a, y2_i = _simple_call(
        _mid_body, [sds2, sds2, sds2, sds2],
        h1a_f, h1a_ta, h1a_i, gamma, beta, W2, dis_f, dis_t, dis_i)
    y2_ta = jnp.concatenate([y2_t, y2_a], axis=1)

    h1_f = _gcn_mmT(adj_fb, y2_f, dis_f, b2r)
    h1_ta = _gcn_mmT(adj_tb, y2_ta, dis_t, jnp.concatenate([b2r, b2r], axis=1))
    h1_i = _gcn_mmT(adj_ib, y2_i, dis_i, b2r)
    h1_t = h1_ta[:, :dout]
    h1_a = h1_ta[:, dout:]

    g = _readout(adj_tb, h1_t, rs_t)

    col = jax.ShapeDtypeStruct((n, 1), F32)
    sc1, sc2, hiden_emb, att_f, att_t, att_i = _simple_call(
        _att_body,
        [col, col, jax.ShapeDtypeStruct((n, dout), F32), col, col, col],
        h1_f, h1_t, h1_a, h1_i, g, wb0, bbr, Wa1, ba1r, wa2r)
    ret = jnp.concatenate([sc1, sc2], axis=1)

    adj123, cs123, rs123 = _weighted_adj(adj_fb, adj_tb, adj_ib,
                                         att_f, att_t, att_i)

    y3, dis_w = _simple_call(
        _dec_in_body,
        [jax.ShapeDtypeStruct((n, dh), BF16),
         jax.ShapeDtypeStruct((n, 1), F32)],
        cs123.reshape(n, 1), rs123, hiden_emb, W3)

    h2a = _sym_mm(adj123, y3, dis_w, b3r, relu=False)

    y4 = _simple_call(
        _dec_mid_body, jax.ShapeDtypeStruct((n, din), BF16),
        h2a, gamma, beta, W4, dis_w)

    h2 = _sym_mm(adj123, y4, dis_w, b4r, relu=True)

    return (hiden_emb, h2, ret)
